# baseline (device time: 432318 ns/iter reference)
import jax
import jax.numpy as jnp
from jax import lax
from jax.experimental import pallas as pl
from jax.experimental.pallas import tpu as pltpu

N_DEV = 32
E_LOC = 4
N_EXP = 128
D_MODEL = 256
D_HID = 512
N_TOK = 1024
DEPTH = 6


def kernel(x, router_W, route_idx, expert_W):
    assert x.shape == (N_TOK, D_MODEL), x.shape
    assert expert_W.shape == (E_LOC, D_MODEL, D_HID), expert_W.shape

    def body(x_ref, rw_ref, idx_ref, ew_ref, out_ref,
             comm_r, comm_l, send_r, recv_r, send_l, recv_l,
             cred_r, cred_l):
        my = lax.axis_index("i")
        left = lax.rem(my - 1 + N_DEV, N_DEV)
        right = lax.rem(my + 1, N_DEV)

        bsem = pltpu.get_barrier_semaphore()
        for nbr in (left, right):
            pl.semaphore_signal(bsem, inc=1, device_id=(nbr,),
                                device_id_type=pl.DeviceIdType.MESH)
        pl.semaphore_wait(bsem, 2)

        xv = x_ref[...]
        scores = jnp.dot(xv, rw_ref[...], preferred_element_type=jnp.float32)
        smax = jnp.max(scores, axis=1, keepdims=True)
        p = jnp.exp(scores - smax)
        p = p / jnp.sum(p, axis=1, keepdims=True)
        idx0 = idx_ref[:, 0:1]
        idx1 = idx_ref[:, 1:2]
        eids = lax.broadcasted_iota(jnp.int32, (N_TOK, N_EXP), 1)
        g0 = jnp.sum(jnp.where(eids == idx0, p, 0.0), axis=1, keepdims=True)
        g1 = jnp.sum(jnp.where(eids == idx1, p, 0.0), axis=1, keepdims=True)
        gs = g0 + g1
        w0 = g0 / gs
        w1 = g1 / gs

        ew = ew_ref[...]
        comm_r[0] = ew[0:2].reshape(2 * D_MODEL, D_HID).astype(jnp.bfloat16)
        comm_l[0] = ew[2:4].reshape(2 * D_MODEL, D_HID).astype(jnp.bfloat16)

        def masked_x(origin, experts):
            parts = []
            for e in experts:
                gid = origin * E_LOC + e
                m = (jnp.where(idx0 == gid, w0, 0.0)
                     + jnp.where(idx1 == gid, w1, 0.0))
                parts.append((xv * m).astype(jnp.bfloat16))
            return jnp.concatenate(parts, axis=1)

        rd_r = [None] * N_DEV
        rd_l = [None] * N_DEV
        acc = jnp.zeros((N_TOK, D_HID), jnp.float32)
        for h in range(N_DEV):
            c = h % DEPTH
            r = (h + 1) % DEPTH
            if h < N_DEV - 1:
                if h >= DEPTH - 1:
                    pl.semaphore_wait(cred_r, 1)
                    pl.semaphore_wait(cred_l, 1)
                rd_r[h] = pltpu.make_async_remote_copy(
                    src_ref=comm_r.at[c], dst_ref=comm_r.at[r],
                    send_sem=send_r.at[c], recv_sem=recv_r.at[r],
                    device_id=(right,),
                    device_id_type=pl.DeviceIdType.MESH,
                )
                rd_r[h].start()
                rd_l[h] = pltpu.make_async_remote_copy(
                    src_ref=comm_l.at[c], dst_ref=comm_l.at[r],
                    send_sem=send_l.at[c], recv_sem=recv_l.at[r],
                    device_id=(left,),
                    device_id_type=pl.DeviceIdType.MESH,
                )
                rd_l[h].start()

            origin_r = lax.rem(my - h + N_DEV, N_DEV)
            origin_l = lax.rem(my + h, N_DEV)
            acc = acc + jnp.dot(masked_x(origin_r, (0, 1)), comm_r[c],
                                preferred_element_type=jnp.float32)
            acc = acc + jnp.dot(masked_x(origin_l, (2, 3)), comm_l[c],
                                preferred_element_type=jnp.float32)

            if h < N_DEV - 1:
                rd_r[h].wait_recv()
                rd_l[h].wait_recv()
            if h >= 2:
                rd_r[h - 2].wait_send()
                rd_l[h - 2].wait_send()
                if h <= N_DEV - 5:
                    pl.semaphore_signal(cred_r, inc=1, device_id=(left,),
                                        device_id_type=pl.DeviceIdType.MESH)
                    pl.semaphore_signal(cred_l, inc=1, device_id=(right,),
                                        device_id_type=pl.DeviceIdType.MESH)

        rd_r[N_DEV - 2].wait_send()
        rd_l[N_DEV - 2].wait_send()

        out_ref[...] = acc

    return pl.pallas_call(
        body,
        out_shape=jax.ShapeDtypeStruct((N_TOK, D_HID), jnp.float32),
        in_specs=[
            pl.BlockSpec(memory_space=pltpu.VMEM),
            pl.BlockSpec(memory_space=pltpu.VMEM),
            pl.BlockSpec(memory_space=pltpu.VMEM),
            pl.BlockSpec(memory_space=pltpu.VMEM),
        ],
        out_specs=pl.BlockSpec(memory_space=pltpu.VMEM),
        scratch_shapes=[
            pltpu.VMEM((DEPTH, 2 * D_MODEL, D_HID), jnp.bfloat16),
            pltpu.VMEM((DEPTH, 2 * D_MODEL, D_HID), jnp.bfloat16),
            pltpu.SemaphoreType.DMA((DEPTH,)),
            pltpu.SemaphoreType.DMA((DEPTH,)),
            pltpu.SemaphoreType.DMA((DEPTH,)),
            pltpu.SemaphoreType.DMA((DEPTH,)),
            pltpu.SemaphoreType.REGULAR,
            pltpu.SemaphoreType.REGULAR,
        ],
        compiler_params=pltpu.CompilerParams(collective_id=0),
    )(x, router_W, route_idx, expert_W)


# device time: 153918 ns/iter; 2.8088x vs baseline; 2.8088x over previous
import jax
import jax.numpy as jnp
from jax import lax
from jax.experimental import pallas as pl
from jax.experimental.pallas import tpu as pltpu

N_DEV = 32
E_LOC = 4
N_EXP = 128
D_MODEL = 256
D_HID = 512
N_TOK = 1024
C = 48
BLK = E_LOC * C
COLS = N_EXP * C


def kernel(x, router_W, route_idx, expert_W):
    assert x.shape == (N_TOK, D_MODEL), x.shape
    assert expert_W.shape == (E_LOC, D_MODEL, D_HID), expert_W.shape

    def body(x_ref, rw_ref, idx_ref, ew_ref, out_ref,
             disp_ref, recv_ref, ret_ref, retr_ref,
             dsend, drecv, rsend, rrecv):
        my = lax.axis_index("i")

        bsem = pltpu.get_barrier_semaphore()
        for dd in range(1, N_DEV):
            d = lax.rem(my + dd, N_DEV)
            pl.semaphore_signal(bsem, inc=1, device_id=(d,),
                                device_id_type=pl.DeviceIdType.MESH)
        pl.semaphore_wait(bsem, N_DEV - 1)

        xv = x_ref[...]
        xb = xv.astype(jnp.bfloat16)
        scores = jnp.dot(xv, rw_ref[...], preferred_element_type=jnp.float32)
        smax = jnp.max(scores, axis=1, keepdims=True)
        p = jnp.exp(scores - smax)
        p = p / jnp.sum(p, axis=1, keepdims=True)
        idx0 = idx_ref[:, 0:1]
        idx1 = idx_ref[:, 1:2]
        eids = lax.broadcasted_iota(jnp.int32, (N_TOK, N_EXP), 1)
        g0 = jnp.sum(jnp.where(eids == idx0, p, 0.0), axis=1, keepdims=True)
        g1 = jnp.sum(jnp.where(eids == idx1, p, 0.0), axis=1, keepdims=True)
        gs = g0 + g1
        w0 = g0 / gs
        w1 = g1 / gs

        eids_t = lax.broadcasted_iota(jnp.int32, (N_EXP, N_TOK), 0)
        idx0_t = idx0.reshape(1, N_TOK)
        idx1_t = idx1.reshape(1, N_TOK)
        cmp0 = eids_t == idx0_t
        cmp1 = eids_t == idx1_t
        pair = cmp0.astype(jnp.int32) + cmp1.astype(jnp.int32)
        cp = pair
        sh = 1
        while sh < N_TOK:
            cp = cp + jnp.concatenate(
                [jnp.zeros((N_EXP, sh), jnp.int32), cp[:, :-sh]], axis=1)
            sh *= 2
        cp = cp - pair

        kio = lax.broadcasted_iota(jnp.int32, (N_EXP, C, N_TOK), 1)
        at_k = cp[:, None, :] == kio
        hit0 = (cmp0[:, None, :] & at_k).astype(jnp.bfloat16)
        hit1 = (cmp1[:, None, :] & at_k).astype(jnp.bfloat16)
        u_t = (hit0 + hit1).reshape(COLS, N_TOK)
        w0_t = w0.astype(jnp.bfloat16).reshape(1, 1, N_TOK)
        w1_t = w1.astype(jnp.bfloat16).reshape(1, 1, N_TOK)
        sw_t = (hit0 * w0_t + hit1 * w1_t).reshape(COLS, N_TOK)

        disp = jnp.dot(u_t, xb, preferred_element_type=jnp.float32)
        disp_ref[...] = disp.astype(jnp.bfloat16).reshape(N_DEV, BLK, D_MODEL)

        d_rdmas = []
        for dd in range(1, N_DEV):
            d = lax.rem(my + dd, N_DEV)
            rd = pltpu.make_async_remote_copy(
                src_ref=disp_ref.at[d], dst_ref=recv_ref.at[my],
                send_sem=dsend.at[dd], recv_sem=drecv.at[dd],
                device_id=(d,), device_id_type=pl.DeviceIdType.MESH,
            )
            rd.start()
            d_rdmas.append(rd)
        recv_ref[pl.ds(my, 1)] = disp_ref[pl.ds(my, 1)]
        for rd in d_rdmas:
            rd.wait_recv()

        recv = recv_ref[...]
        ew_b = ew_ref[...].astype(jnp.bfloat16)
        for e in range(E_LOC):
            rows = recv[:, e * C:(e + 1) * C, :].reshape(N_DEV * C, D_MODEL)
            res = jnp.dot(rows, ew_b[e], preferred_element_type=jnp.float32)
            ret_ref[:, e * C:(e + 1) * C, :] = (
                res.astype(jnp.bfloat16).reshape(N_DEV, C, D_HID))

        r_rdmas = []
        for dd in range(1, N_DEV):
            s = lax.rem(my + dd, N_DEV)
            rd = pltpu.make_async_remote_copy(
                src_ref=ret_ref.at[s], dst_ref=retr_ref.at[my],
                send_sem=rsend.at[dd], recv_sem=rrecv.at[dd],
                device_id=(s,), device_id_type=pl.DeviceIdType.MESH,
            )
            rd.start()
            r_rdmas.append(rd)
        retr_ref[pl.ds(my, 1)] = ret_ref[pl.ds(my, 1)]
        for rd in r_rdmas:
            rd.wait_recv()

        retr = retr_ref[...].reshape(COLS, D_HID)
        out_ref[...] = lax.dot_general(
            sw_t, retr, (((0,), (0,)), ((), ())),
            preferred_element_type=jnp.float32)

        for rd in d_rdmas:
            rd.wait_send()
        for rd in r_rdmas:
            rd.wait_send()

    return pl.pallas_call(
        body,
        out_shape=jax.ShapeDtypeStruct((N_TOK, D_HID), jnp.float32),
        in_specs=[
            pl.BlockSpec(memory_space=pltpu.VMEM),
            pl.BlockSpec(memory_space=pltpu.VMEM),
            pl.BlockSpec(memory_space=pltpu.VMEM),
            pl.BlockSpec(memory_space=pltpu.VMEM),
        ],
        out_specs=pl.BlockSpec(memory_space=pltpu.VMEM),
        scratch_shapes=[
            pltpu.VMEM((N_DEV, BLK, D_MODEL), jnp.bfloat16),
            pltpu.VMEM((N_DEV, BLK, D_MODEL), jnp.bfloat16),
            pltpu.VMEM((N_DEV, BLK, D_HID), jnp.bfloat16),
            pltpu.VMEM((N_DEV, BLK, D_HID), jnp.bfloat16),
            pltpu.SemaphoreType.DMA((N_DEV,)),
            pltpu.SemaphoreType.DMA((N_DEV,)),
            pltpu.SemaphoreType.DMA((N_DEV,)),
            pltpu.SemaphoreType.DMA((N_DEV,)),
        ],
        compiler_params=pltpu.CompilerParams(
            collective_id=0, vmem_limit_bytes=110 * 1024 * 1024),
    )(x, router_W, route_idx, expert_W)
